# Initial kernel scaffold; baseline (speedup 1.0000x reference)
import functools

import jax
import jax.numpy as jnp
from jax import lax
from jax.experimental import pallas as pl
from jax.experimental.pallas import tpu as pltpu
from jax.experimental.pallas import tpu_sc as plsc

N_NODES = 100000
N_EDGES = 3200000
D = 16

NC, NS = 2, 16
NW = NC * NS
EDGES_PER_TILE = N_EDGES // NW
K = 80
ROWS_PER_LOAD = 25
EPL = K * ROWS_PER_LOAD
LOADS = EDGES_PER_TILE // EPL
STRIPE = N_NODES // NS
ZROWS = 625

_mesh = plsc.VectorSubcoreMesh(core_axis_name="c", subcore_axis_name="s")


@functools.partial(
    pl.kernel,
    out_type=[
        jax.ShapeDtypeStruct((N_NODES, D), jnp.float32),
        jax.ShapeDtypeStruct((N_NODES, D), jnp.float32),
    ],
    mesh=_mesh,
    scratch_types=[
        pltpu.VMEM_SHARED((N_NODES, D), jnp.float32),
        pltpu.VMEM((2, ROWS_PER_LOAD, K), jnp.int32),
        pltpu.VMEM((2, EPL, D), jnp.float32),
        pltpu.VMEM((ZROWS, D), jnp.float32),
        pltpu.SemaphoreType.DMA,
        pltpu.SemaphoreType.DMA,
        pltpu.SemaphoreType.DMA,
        pltpu.SemaphoreType.DMA,
    ],
)
def _scatter_sc(attr_hbm, idx_hbm, out0, out1,
                acc, idx_v, attr_v, zer_v, is0, is1, as0, as1):
    cid = lax.axis_index("c")
    sid = lax.axis_index("s")
    wid = sid * NC + cid
    isems = (is0, is1)
    asems = (as0, as1)

    def _z(i, c):
        zer_v[i, :] = jnp.zeros((D,), jnp.float32)
        return c
    lax.fori_loop(0, ZROWS, _z, 0)
    r0 = sid * STRIPE
    for kk in range(STRIPE // ZROWS):
        pltpu.sync_copy(zer_v, acc.at[pl.ds(r0 + kk * ZROWS, ZROWS)])
    plsc.subcore_barrier()

    row_base = wid * (EDGES_PER_TILE // K)
    e_base = wid * EDGES_PER_TILE

    def _fire(b, g):
        pltpu.async_copy(
            idx_hbm.at[pl.ds(row_base + g * ROWS_PER_LOAD, ROWS_PER_LOAD)],
            idx_v.at[b], isems[b])
        pltpu.async_copy(
            attr_hbm.at[pl.ds(e_base + g * EPL, EPL)],
            attr_v.at[b], asems[b])

    def _wait(b):
        pltpu.make_async_copy(
            idx_hbm.at[pl.ds(0, ROWS_PER_LOAD)], idx_v.at[b], isems[b]).wait()
        pltpu.make_async_copy(
            attr_hbm.at[pl.ds(0, EPL)], attr_v.at[b], asems[b]).wait()

    _fire(0, 0)

    def _outer(g2, c):
        for b in range(2):
            g = g2 * 2 + b

            @pl.when(g + 1 < LOADS)
            def _prefetch():
                _fire(1 - b, g + 1)

            _wait(b)

            def _scat(j, cc):
                pltpu.sync_copy(attr_v.at[b, pl.ds(j * K, K)],
                                acc.at[idx_v.at[b, j]], add=True)
                return cc
            lax.fori_loop(0, ROWS_PER_LOAD, _scat, 0)
        return c
    lax.fori_loop(0, LOADS // 2, _outer, 0)

    plsc.subcore_barrier()

    @pl.when(cid == 0)
    def _w0():
        pltpu.sync_copy(acc.at[pl.ds(r0, STRIPE)], out0.at[pl.ds(r0, STRIPE)])

    @pl.when(cid == 1)
    def _w1():
        pltpu.sync_copy(acc.at[pl.ds(r0, STRIPE)], out1.at[pl.ds(r0, STRIPE)])


def _add_body(a_ref, b_ref, o_ref):
    o_ref[...] = a_ref[...] + b_ref[...]


def _tc_add(a, b):
    rows, cols = a.shape
    g = 10
    blk = rows // g
    return pl.pallas_call(
        _add_body,
        grid=(g,),
        in_specs=[pl.BlockSpec((blk, cols), lambda i: (i, 0))] * 2,
        out_specs=pl.BlockSpec((blk, cols), lambda i: (i, 0)),
        out_shape=jax.ShapeDtypeStruct((rows, cols), jnp.float32),
    )(a, b)


def kernel(edge_attr, edge_index):
    idx2d = edge_index[1].astype(jnp.int32).reshape(N_EDGES // K, K)
    p0, p1 = _scatter_sc(edge_attr, idx2d)
    a = p0.reshape(N_NODES * D // 128, 128)
    b = p1.reshape(N_NODES * D // 128, 128)
    return _tc_add(a, b).reshape(N_NODES, D)


# speedup vs baseline: 6.5555x; 6.5555x over previous
import functools

import jax
import jax.numpy as jnp
from jax import lax
from jax.experimental import pallas as pl
from jax.experimental.pallas import tpu as pltpu
from jax.experimental.pallas import tpu_sc as plsc

N_NODES = 100000
N_EDGES = 3200000
D = 16

NC, NS = 2, 16
NW = NC * NS
EDGES_PER_TILE = N_EDGES // NW
K = 80
ROWS_PER_LOAD = 5
EPL = K * ROWS_PER_LOAD
LOADS = EDGES_PER_TILE // EPL
STRIPE = N_NODES // NS
ZROWS = 125

_mesh = plsc.VectorSubcoreMesh(core_axis_name="c", subcore_axis_name="s")


@functools.partial(
    pl.kernel,
    out_type=[
        jax.ShapeDtypeStruct((N_NODES, D), jnp.float32),
        jax.ShapeDtypeStruct((N_NODES, D), jnp.float32),
    ],
    mesh=_mesh,
    compiler_params=pltpu.CompilerParams(use_tc_tiling_on_sc=False),
    scratch_types=[
        pltpu.VMEM_SHARED((N_NODES, D), jnp.float32),
        pltpu.VMEM((2, ROWS_PER_LOAD, K), jnp.int32),
        pltpu.VMEM((2, EPL, D), jnp.float32),
        pltpu.VMEM((ZROWS, D), jnp.float32),
        pltpu.SemaphoreType.DMA,
        pltpu.SemaphoreType.DMA,
        pltpu.SemaphoreType.DMA,
        pltpu.SemaphoreType.DMA,
    ],
)
def _scatter_sc(attr_hbm, idx_hbm, out0, out1,
                acc, idx_v, attr_v, zer_v, is0, is1, as0, as1):
    cid = lax.axis_index("c")
    sid = lax.axis_index("s")
    wid = sid * NC + cid
    isems = (is0, is1)
    asems = (as0, as1)

    def _z(i, c):
        zer_v[i, :] = jnp.zeros((D,), jnp.float32)
        return c
    lax.fori_loop(0, ZROWS, _z, 0)
    r0 = sid * STRIPE
    for kk in range(STRIPE // ZROWS):
        pltpu.sync_copy(zer_v, acc.at[pl.ds(r0 + kk * ZROWS, ZROWS)])
    plsc.subcore_barrier()

    row_base = wid * (EDGES_PER_TILE // K)
    e_base = wid * EDGES_PER_TILE

    def _fire(b, g):
        pltpu.async_copy(
            idx_hbm.at[pl.ds(row_base + g * ROWS_PER_LOAD, ROWS_PER_LOAD)],
            idx_v.at[b], isems[b])
        pltpu.async_copy(
            attr_hbm.at[pl.ds(e_base + g * EPL, EPL)],
            attr_v.at[b], asems[b])

    def _wait(b):
        pltpu.make_async_copy(
            idx_hbm.at[pl.ds(0, ROWS_PER_LOAD)], idx_v.at[b], isems[b]).wait()
        pltpu.make_async_copy(
            attr_hbm.at[pl.ds(0, EPL)], attr_v.at[b], asems[b]).wait()

    _fire(0, 0)

    def _outer(g2, c):
        for b in range(2):
            g = g2 * 2 + b

            @pl.when(g + 1 < LOADS)
            def _prefetch():
                _fire(1 - b, g + 1)

            _wait(b)

            def _scat(j, cc):
                pltpu.sync_copy(attr_v.at[b, pl.ds(j * K, K)],
                                acc.at[idx_v.at[b, j]], add=True)
                return cc
            lax.fori_loop(0, ROWS_PER_LOAD, _scat, 0)
        return c
    lax.fori_loop(0, LOADS // 2, _outer, 0)

    plsc.subcore_barrier()

    @pl.when(cid == 0)
    def _w0():
        pltpu.sync_copy(acc.at[pl.ds(r0, STRIPE)], out0.at[pl.ds(r0, STRIPE)])

    @pl.when(cid == 1)
    def _w1():
        pltpu.sync_copy(acc.at[pl.ds(r0, STRIPE)], out1.at[pl.ds(r0, STRIPE)])


def _add_body(a_ref, b_ref, o_ref):
    o_ref[...] = a_ref[...] + b_ref[...]


def _tc_add(a, b):
    rows, cols = a.shape
    return pl.pallas_call(
        _add_body,
        out_shape=jax.ShapeDtypeStruct((rows, cols), jnp.float32),
    )(a, b)


def kernel(edge_attr, edge_index):
    idx2d = edge_index[1].astype(jnp.int32).reshape(N_EDGES // K, K)
    p0, p1 = _scatter_sc(edge_attr, idx2d)
    a = p0.reshape(N_NODES * D // 128, 128)
    b = p1.reshape(N_NODES * D // 128, 128)
    return _tc_add(a, b).reshape(N_NODES, D)


# speedup vs baseline: 6.8831x; 1.0500x over previous
import functools

import jax
import jax.numpy as jnp
from jax import lax
from jax.experimental import pallas as pl
from jax.experimental.pallas import tpu as pltpu
from jax.experimental.pallas import tpu_sc as plsc

N_NODES = 100000
N_EDGES = 3200000
D = 16

NC, NS = 2, 16
NW = NC * NS
EDGES_PER_TILE = N_EDGES // NW
EPL = 800
LOADS = EDGES_PER_TILE // EPL
STRIPE = N_NODES // NS
ZROWS = 125

_mesh = plsc.VectorSubcoreMesh(core_axis_name="c", subcore_axis_name="s")


@functools.partial(
    pl.kernel,
    out_type=[
        jax.ShapeDtypeStruct((N_NODES, D), jnp.float32),
        jax.ShapeDtypeStruct((N_NODES, D), jnp.float32),
    ],
    mesh=_mesh,
    compiler_params=pltpu.CompilerParams(use_tc_tiling_on_sc=False),
    scratch_types=[
        pltpu.VMEM_SHARED((N_NODES, D), jnp.float32),
        pltpu.VMEM((2, EPL), jnp.int32),
        pltpu.VMEM((2, EPL, D), jnp.float32),
        pltpu.VMEM((ZROWS, D), jnp.float32),
        pltpu.SemaphoreType.DMA,
        pltpu.SemaphoreType.DMA,
        pltpu.SemaphoreType.DMA,
        pltpu.SemaphoreType.DMA,
        pltpu.SemaphoreType.DMA,
        pltpu.SemaphoreType.DMA,
    ],
)
def _scatter_sc(attr_hbm, idx_hbm, out0, out1,
                acc, idx_v, attr_v, zer_v, is0, is1, as0, as1, ss0, ss1):
    cid = lax.axis_index("c")
    sid = lax.axis_index("s")
    wid = sid * NC + cid
    isems = (is0, is1)
    asems = (as0, as1)
    ssems = (ss0, ss1)

    def _z(i, c):
        zer_v[i, :] = jnp.zeros((D,), jnp.float32)
        return c
    lax.fori_loop(0, ZROWS, _z, 0)
    r0 = sid * STRIPE
    for kk in range(STRIPE // ZROWS):
        pltpu.sync_copy(zer_v, acc.at[pl.ds(r0 + kk * ZROWS, ZROWS)])
    plsc.subcore_barrier()

    e_base = wid * EDGES_PER_TILE

    def fire_load(b, g):
        pltpu.async_copy(idx_hbm.at[pl.ds(e_base + g * EPL, EPL)],
                         idx_v.at[b], isems[b])
        pltpu.async_copy(attr_hbm.at[pl.ds(e_base + g * EPL, EPL)],
                         attr_v.at[b], asems[b])

    def wait_load(b):
        pltpu.make_async_copy(idx_hbm.at[pl.ds(0, EPL)],
                              idx_v.at[b], isems[b]).wait()
        pltpu.make_async_copy(attr_hbm.at[pl.ds(0, EPL)],
                              attr_v.at[b], asems[b]).wait()

    def fire_scat(b):
        pltpu.async_copy(attr_v.at[b], acc.at[idx_v.at[b]], ssems[b],
                         add=True)

    def wait_scat(b):
        pltpu.make_async_copy(attr_v.at[b], acc.at[idx_v.at[b]],
                              ssems[b]).wait()

    fire_load(0, 0)
    fire_load(1, 1)
    wait_load(0)
    fire_scat(0)

    def _pair(g2, c):
        for b in (1, 0):
            g = 2 * g2 + (1 if b == 1 else 2)
            wait_scat(1 - b)

            @pl.when(g + 1 < LOADS)
            def _prefetch():
                fire_load(1 - b, g + 1)

            wait_load(b)
            fire_scat(b)
        return c
    lax.fori_loop(0, (LOADS - 1) // 2, _pair, 0)
    wait_scat(0)

    plsc.subcore_barrier()

    @pl.when(cid == 0)
    def _w0():
        pltpu.sync_copy(acc.at[pl.ds(r0, STRIPE)], out0.at[pl.ds(r0, STRIPE)])

    @pl.when(cid == 1)
    def _w1():
        pltpu.sync_copy(acc.at[pl.ds(r0, STRIPE)], out1.at[pl.ds(r0, STRIPE)])


def _add_body(a_ref, b_ref, o_ref):
    o_ref[...] = a_ref[...] + b_ref[...]


def _tc_add(a, b):
    rows, cols = a.shape
    return pl.pallas_call(
        _add_body,
        out_shape=jax.ShapeDtypeStruct((rows, cols), jnp.float32),
    )(a, b)


def kernel(edge_attr, edge_index):
    idx1 = edge_index[1].astype(jnp.int32)
    p0, p1 = _scatter_sc(edge_attr, idx1)
    a = p0.reshape(N_NODES * D // 128, 128)
    b = p1.reshape(N_NODES * D // 128, 128)
    return _tc_add(a, b).reshape(N_NODES, D)


# speedup vs baseline: 7.9489x; 1.1548x over previous
import functools

import jax
import jax.numpy as jnp
from jax import lax
from jax.experimental import pallas as pl
from jax.experimental.pallas import tpu as pltpu
from jax.experimental.pallas import tpu_sc as plsc

N_NODES = 100000
N_EDGES = 3200000
D = 16

NC, NS = 2, 16
NW = NC * NS
EDGES_PER_TILE = N_EDGES // NW
EPL = 800
LOADS = EDGES_PER_TILE // EPL
STRIPE = N_NODES // NS
ZROWS = 125

_mesh = plsc.VectorSubcoreMesh(core_axis_name="c", subcore_axis_name="s")


@functools.partial(
    pl.kernel,
    out_type=[
        jax.ShapeDtypeStruct((N_NODES, D), jnp.float32),
        jax.ShapeDtypeStruct((N_NODES, D), jnp.float32),
    ],
    mesh=_mesh,
    compiler_params=pltpu.CompilerParams(use_tc_tiling_on_sc=False),
    scratch_types=[
        pltpu.VMEM_SHARED((N_NODES, D), jnp.float32),
        pltpu.VMEM((2, EPL), jnp.int32),
        pltpu.VMEM((2, EPL, D), jnp.float32),
        pltpu.VMEM((ZROWS, D), jnp.float32),
        pltpu.SemaphoreType.DMA,
        pltpu.SemaphoreType.DMA,
        pltpu.SemaphoreType.DMA,
        pltpu.SemaphoreType.DMA,
        pltpu.SemaphoreType.DMA,
        pltpu.SemaphoreType.DMA,
    ],
)
def _scatter_sc(attr_hbm, idx_hbm, out0, out1,
                acc, idx_v, attr_v, zer_v, is0, is1, as0, as1, ss0, ss1):
    cid = lax.axis_index("c")
    sid = lax.axis_index("s")
    wid = sid * NC + cid
    isems = (is0, is1)
    asems = (as0, as1)
    ssems = (ss0, ss1)

    def _z(i, c):
        zer_v[i, :] = jnp.zeros((D,), jnp.float32)
        return c
    lax.fori_loop(0, ZROWS, _z, 0)
    r0 = sid * STRIPE
    for kk in range(STRIPE // ZROWS):
        pltpu.sync_copy(zer_v, acc.at[pl.ds(r0 + kk * ZROWS, ZROWS)])
    plsc.subcore_barrier()

    e_base = wid * EDGES_PER_TILE

    def fire_load(b, g):
        pltpu.async_copy(idx_hbm.at[pl.ds(e_base + g * EPL, EPL)],
                         idx_v.at[b], isems[b])
        pltpu.async_copy(attr_hbm.at[pl.ds(e_base + g * EPL, EPL)],
                         attr_v.at[b], asems[b])

    def wait_load(b):
        pltpu.make_async_copy(idx_hbm.at[pl.ds(0, EPL)],
                              idx_v.at[b], isems[b]).wait()
        pltpu.make_async_copy(attr_hbm.at[pl.ds(0, EPL)],
                              attr_v.at[b], asems[b]).wait()

    def fire_scat(b):
        pltpu.async_copy(attr_v.at[b], acc.at[idx_v.at[b]], ssems[b],
                         add=True)

    def wait_scat(b):
        pltpu.make_async_copy(attr_v.at[b], acc.at[idx_v.at[b]],
                              ssems[b]).wait()

    fire_load(0, 0)
    fire_load(1, 1)
    wait_load(0)
    fire_scat(0)

    def _pair(g2, c):
        for b in (1, 0):
            g = 2 * g2 + (1 if b == 1 else 2)
            wait_scat(1 - b)

            @pl.when(g + 1 < LOADS)
            def _prefetch():
                fire_load(1 - b, g + 1)

            wait_load(b)
            fire_scat(b)
        return c
    lax.fori_loop(0, (LOADS - 1) // 2, _pair, 0)
    wait_scat(0)

    plsc.subcore_barrier()

    @pl.when(cid == 0)
    def _w0():
        pltpu.sync_copy(acc.at[pl.ds(r0, STRIPE)], out0.at[pl.ds(r0, STRIPE)])

    @pl.when(cid == 1)
    def _w1():
        pltpu.sync_copy(acc.at[pl.ds(r0, STRIPE)], out1.at[pl.ds(r0, STRIPE)])


def _tr_body(a_ref, o_ref):
    t = a_ref[...].T.reshape(a_ref.shape[1] // 8, 8, D)
    for q in range(8):
        o_ref[:, 16 * q:16 * (q + 1)] = t[:, q, :]


def _tc_transpose(at):
    chunk = 3200
    grid = N_EDGES // chunk
    orows = chunk * D // 128
    return pl.pallas_call(
        _tr_body,
        grid=(grid,),
        in_specs=[pl.BlockSpec((D, chunk), lambda i: (0, i))],
        out_specs=pl.BlockSpec((orows, 128), lambda i: (i, 0)),
        out_shape=jax.ShapeDtypeStruct((N_EDGES * D // 128, 128),
                                       jnp.float32),
    )(at)


def _add_body(a_ref, b_ref, o_ref):
    o_ref[...] = a_ref[...] + b_ref[...]


def _tc_add(a, b):
    rows, cols = a.shape
    return pl.pallas_call(
        _add_body,
        out_shape=jax.ShapeDtypeStruct((rows, cols), jnp.float32),
    )(a, b)


def kernel(edge_attr, edge_index):
    attr_lin = _tc_transpose(edge_attr.T)
    attr2d = attr_lin.reshape(N_EDGES, D)
    idx1 = edge_index[1].astype(jnp.int32)
    p0, p1 = _scatter_sc(attr2d, idx1)
    a = p0.reshape(N_NODES * D // 128, 128)
    b = p1.reshape(N_NODES * D // 128, 128)
    return _tc_add(a, b).reshape(N_NODES, D)
